# Initial kernel scaffold; baseline (speedup 1.0000x reference)
import jax
import jax.numpy as jnp
from jax.experimental import pallas as pl


def kernel(x, pos_weight):
    raise NotImplementedError("write your pallas kernel here")


# speedup vs baseline: 1.7290x; 1.7290x over previous
import jax
import jax.numpy as jnp
from jax.experimental import pallas as pl


def _add_body(x_ref, p_ref, o_ref):
    o_ref[...] = x_ref[...] + p_ref[...]


def kernel(x, pos_weight):
    B, T, D = x.shape
    BLK = 512
    return pl.pallas_call(
        _add_body,
        grid=(T // BLK,),
        in_specs=[
            pl.BlockSpec((B, BLK, D), lambda t: (0, t, 0)),
            pl.BlockSpec((BLK, D), lambda t: (t, 0)),
        ],
        out_specs=pl.BlockSpec((B, BLK, D), lambda t: (0, t, 0)),
        out_shape=jax.ShapeDtypeStruct(x.shape, x.dtype),
    )(x, pos_weight)
